# baseline (speedup 1.0000x reference)
import jax
import jax.numpy as jnp
from jax.experimental import pallas as pl
from jax.experimental.pallas import tpu as pltpu

_PAD = 64


def _router_kernel(x_ref, w1_ref, b1_ref, w2_ref, b2_ref,
                   probs_ref, idx_ref, val_ref):
    B, C, HW = x_ref.shape
    ones = jnp.ones((HW, 1), jnp.float32)
    sums = [jnp.dot(x_ref[b], ones, preferred_element_type=jnp.float32)
            for b in range(B)]
    mT = jnp.concatenate(sums, axis=1) * (1.0 / HW)
    zT = jnp.maximum(
        jnp.dot(w1_ref[...], mT, preferred_element_type=jnp.float32)
        + b1_ref[...], 0.0)
    logitsT = jnp.dot(w2_ref[...], zT,
                      preferred_element_type=jnp.float32) + b2_ref[...]
    lmax = jnp.max(logitsT, axis=0, keepdims=True)
    ex = jnp.exp(logitsT - lmax)
    probsT = ex / jnp.sum(ex, axis=0, keepdims=True)
    E = probsT.shape[0]
    row = jax.lax.broadcasted_iota(jnp.int32, probsT.shape, 0)
    v1 = jnp.max(probsT, axis=0, keepdims=True)
    i1 = jnp.min(jnp.where(probsT == v1, row, E), axis=0, keepdims=True)
    masked = jnp.where(row == i1, -jnp.inf, probsT)
    v2 = jnp.max(masked, axis=0, keepdims=True)
    i2 = jnp.min(jnp.where(masked == v2, row, E), axis=0, keepdims=True)
    probs_ref[...] = probsT.T
    idx_ref[...] = jnp.concatenate([i1, i2], axis=0).T
    val_ref[...] = jnp.concatenate([v1, v2], axis=0).T


def _moe_kernel(idx_ref, val_ref, x_ref, ml_ref, mr_ref, w0_ref, w1_ref,
                beta0_ref, beta1_ref, out_ref, xc_ref, xl_ref, xr_ref):
    b = pl.program_id(0)
    HW, C = x_ref.shape[1], x_ref.shape[2]
    W = 56
    xx = x_ref[0]

    zrow = jnp.zeros((_PAD, C), jnp.float32)
    sh_l = jnp.concatenate([jnp.zeros((1, C), jnp.float32), xx[:-1, :]],
                           axis=0)
    sh_r = jnp.concatenate([xx[1:, :], jnp.zeros((1, C), jnp.float32)],
                           axis=0)
    for ref, mid in ((xc_ref, xx), (xl_ref, sh_l * ml_ref[...]),
                     (xr_ref, sh_r * mr_ref[...])):
        ref[0:_PAD, :] = zrow
        ref[pl.ds(_PAD, HW), :] = mid
        ref[pl.ds(_PAD + HW, _PAD), :] = zrow

    rw0 = val_ref[b, 0]
    rw1 = val_ref[b, 1]
    acc0 = jnp.zeros((HW, C), jnp.float32)
    acc1 = jnp.zeros((HW, C), jnp.float32)
    w0 = w0_ref[0].astype(jnp.float32) * rw0
    w1 = w1_ref[0].astype(jnp.float32) * rw1
    for t in range(9):
        dy, dx = t // 3 - 1, t % 3 - 1
        buf = (xl_ref, xc_ref, xr_ref)[dx + 1]
        sh = buf[pl.ds(_PAD + dy * W, HW), :]
        acc0 = acc0 + jnp.dot(sh, w0[t],
                              preferred_element_type=jnp.float32)
        acc1 = acc1 + jnp.dot(sh, w1[t],
                              preferred_element_type=jnp.float32)
    o0 = jnp.maximum(acc0 + beta0_ref[0] * rw0, 0.0)
    o1 = jnp.maximum(acc1 + beta1_ref[0] * rw1, 0.0)
    out_ref[0] = xx + o0 + o1


def kernel(x, fc1_w, fc1_b, fc2_w, fc2_b, conv_w, bn_gamma, bn_beta):
    B, C, H, W = x.shape
    E, HID = fc2_w.shape[0], fc1_w.shape[0]
    HW = H * W
    K = 2
    xt = x.reshape(B, C, HW).transpose(0, 2, 1)

    probs, idx2, val2 = pl.pallas_call(
        _router_kernel,
        out_shape=[
            jax.ShapeDtypeStruct((B, E), jnp.float32),
            jax.ShapeDtypeStruct((B, K), jnp.int32),
            jax.ShapeDtypeStruct((B, K), jnp.float32),
        ],
    )(x.reshape(B, C, HW), fc1_w, fc1_b.reshape(HID, 1), fc2_w,
      fc2_b.reshape(E, 1))

    eps = 1e-5
    scale = bn_gamma * (1.0 / jnp.sqrt(1.0 + eps))
    wt = (conv_w * scale[:, :, None, None, None]).astype(jnp.bfloat16)
    wt = wt.transpose(0, 3, 4, 2, 1).reshape(E, 9, C, C)
    beta3 = bn_beta.reshape(E, 1, C)
    pos = jnp.arange(HW, dtype=jnp.int32).reshape(HW, 1) % W
    m_l = (pos != 0).astype(jnp.float32)
    m_r = (pos != W - 1).astype(jnp.float32)

    grid_spec = pltpu.PrefetchScalarGridSpec(
        num_scalar_prefetch=2,
        grid=(B,),
        in_specs=[
            pl.BlockSpec((1, HW, C), lambda b, idx, val: (b, 0, 0)),
            pl.BlockSpec((HW, 1), lambda b, idx, val: (0, 0)),
            pl.BlockSpec((HW, 1), lambda b, idx, val: (0, 0)),
            pl.BlockSpec((1, 9, C, C),
                         lambda b, idx, val: (idx[b, 0], 0, 0, 0)),
            pl.BlockSpec((1, 9, C, C),
                         lambda b, idx, val: (idx[b, 1], 0, 0, 0)),
            pl.BlockSpec((1, 1, C),
                         lambda b, idx, val: (idx[b, 0], 0, 0)),
            pl.BlockSpec((1, 1, C),
                         lambda b, idx, val: (idx[b, 1], 0, 0)),
        ],
        out_specs=pl.BlockSpec((1, HW, C), lambda b, idx, val: (b, 0, 0)),
        scratch_shapes=[pltpu.VMEM((2 * _PAD + HW, C), jnp.float32)] * 3,
    )
    out_t = pl.pallas_call(
        _moe_kernel,
        grid_spec=grid_spec,
        out_shape=jax.ShapeDtypeStruct((B, HW, C), jnp.float32),
    )(idx2, val2, xt, m_l, m_r, wt, wt, beta3, beta3)

    out = out_t.transpose(0, 2, 1).reshape(B, C, H, W)
    return (out, probs)


# speedup vs baseline: 1.2311x; 1.2311x over previous
import jax
import jax.numpy as jnp
from jax.experimental import pallas as pl
from jax.experimental.pallas import tpu as pltpu

_PAD = 64


def _router_kernel(x_ref, w1_ref, b1_ref, w2_ref, b2_ref,
                   probs_ref, idx_ref, val_ref):
    B, HW, C = x_ref.shape
    ones = jnp.ones((1, HW), jnp.float32)
    sums = [jnp.dot(ones, x_ref[b], preferred_element_type=jnp.float32)
            for b in range(B)]
    m = jnp.concatenate(sums, axis=0) * (1.0 / HW)
    z = jnp.maximum(
        jnp.dot(m, w1_ref[...], preferred_element_type=jnp.float32)
        + b1_ref[...], 0.0)
    logits = jnp.dot(z, w2_ref[...],
                     preferred_element_type=jnp.float32) + b2_ref[...]
    probs = jax.nn.softmax(logits, axis=1)
    E = probs.shape[1]
    col = jax.lax.broadcasted_iota(jnp.int32, probs.shape, 1)
    v1 = jnp.max(probs, axis=1, keepdims=True)
    i1 = jnp.min(jnp.where(probs == v1, col, E), axis=1, keepdims=True)
    masked = jnp.where(col == i1, -jnp.inf, probs)
    v2 = jnp.max(masked, axis=1, keepdims=True)
    i2 = jnp.min(jnp.where(masked == v2, col, E), axis=1, keepdims=True)
    probs_ref[...] = probs
    idx_ref[...] = jnp.concatenate([i1, i2], axis=1)
    val_ref[...] = jnp.concatenate([v1, v2], axis=1)


def _moe_kernel(idx_ref, val_ref, x_ref, ml_ref, mr_ref, w0_ref, w1_ref,
                beta0_ref, beta1_ref, out_ref, xc_ref, xl_ref, xr_ref):
    b = pl.program_id(0)
    HW, C = x_ref.shape[1], x_ref.shape[2]
    W = 56
    xx = x_ref[0]

    zrow = jnp.zeros((_PAD, C), jnp.float32)
    sh_l = jnp.concatenate([jnp.zeros((1, C), jnp.float32), xx[:-1, :]],
                           axis=0)
    sh_r = jnp.concatenate([xx[1:, :], jnp.zeros((1, C), jnp.float32)],
                           axis=0)
    for ref, mid in ((xc_ref, xx), (xl_ref, sh_l * ml_ref[...]),
                     (xr_ref, sh_r * mr_ref[...])):
        ref[0:_PAD, :] = zrow
        ref[pl.ds(_PAD, HW), :] = mid
        ref[pl.ds(_PAD + HW, _PAD), :] = zrow

    rw0 = val_ref[b, 0]
    rw1 = val_ref[b, 1]
    acc0 = jnp.zeros((HW, C), jnp.float32)
    acc1 = jnp.zeros((HW, C), jnp.float32)
    w0 = w0_ref[0].astype(jnp.float32) * rw0
    w1 = w1_ref[0].astype(jnp.float32) * rw1
    for t in range(9):
        dy, dx = t // 3 - 1, t % 3 - 1
        buf = (xl_ref, xc_ref, xr_ref)[dx + 1]
        sh = buf[pl.ds(_PAD + dy * W, HW), :]
        acc0 = acc0 + jnp.dot(sh, w0[t],
                              preferred_element_type=jnp.float32)
        acc1 = acc1 + jnp.dot(sh, w1[t],
                              preferred_element_type=jnp.float32)
    o0 = jnp.maximum(acc0 + beta0_ref[0] * rw0, 0.0)
    o1 = jnp.maximum(acc1 + beta1_ref[0] * rw1, 0.0)
    out_ref[0] = xx + o0 + o1


def kernel(x, fc1_w, fc1_b, fc2_w, fc2_b, conv_w, bn_gamma, bn_beta):
    B, C, H, W = x.shape
    E, HID = fc2_w.shape[0], fc1_w.shape[0]
    HW = H * W
    K = 2
    xt = x.reshape(B, C, HW).transpose(0, 2, 1)

    probs, idx2, val2 = pl.pallas_call(
        _router_kernel,
        out_shape=[
            jax.ShapeDtypeStruct((B, E), jnp.float32),
            jax.ShapeDtypeStruct((B, K), jnp.int32),
            jax.ShapeDtypeStruct((B, K), jnp.float32),
        ],
    )(xt, fc1_w.T, fc1_b.reshape(1, HID), fc2_w.T, fc2_b.reshape(1, E))

    eps = 1e-5
    scale = bn_gamma * (1.0 / jnp.sqrt(1.0 + eps))
    wt = (conv_w * scale[:, :, None, None, None]).astype(jnp.bfloat16)
    wt = wt.transpose(0, 3, 4, 2, 1).reshape(E, 9, C, C)
    beta3 = bn_beta.reshape(E, 1, C)
    pos = jnp.arange(HW, dtype=jnp.int32).reshape(HW, 1) % W
    m_l = (pos != 0).astype(jnp.float32)
    m_r = (pos != W - 1).astype(jnp.float32)

    grid_spec = pltpu.PrefetchScalarGridSpec(
        num_scalar_prefetch=2,
        grid=(B,),
        in_specs=[
            pl.BlockSpec((1, HW, C), lambda b, idx, val: (b, 0, 0)),
            pl.BlockSpec((HW, 1), lambda b, idx, val: (0, 0)),
            pl.BlockSpec((HW, 1), lambda b, idx, val: (0, 0)),
            pl.BlockSpec((1, 9, C, C),
                         lambda b, idx, val: (idx[b, 0], 0, 0, 0)),
            pl.BlockSpec((1, 9, C, C),
                         lambda b, idx, val: (idx[b, 1], 0, 0, 0)),
            pl.BlockSpec((1, 1, C),
                         lambda b, idx, val: (idx[b, 0], 0, 0)),
            pl.BlockSpec((1, 1, C),
                         lambda b, idx, val: (idx[b, 1], 0, 0)),
        ],
        out_specs=pl.BlockSpec((1, HW, C), lambda b, idx, val: (b, 0, 0)),
        scratch_shapes=[pltpu.VMEM((2 * _PAD + HW, C), jnp.float32)] * 3,
    )
    out_t = pl.pallas_call(
        _moe_kernel,
        grid_spec=grid_spec,
        out_shape=jax.ShapeDtypeStruct((B, HW, C), jnp.float32),
    )(idx2, val2, xt, m_l, m_r, wt, wt, beta3, beta3)

    out = out_t.transpose(0, 2, 1).reshape(B, C, H, W)
    return (out, probs)
